# Initial kernel scaffold; baseline (speedup 1.0000x reference)
import jax
import jax.numpy as jnp
from jax.experimental import pallas as pl


def kernel(x, Wg, bg, W1, b1, W2, b2):
    raise NotImplementedError("write your pallas kernel here")


# speedup vs baseline: 1.6736x; 1.6736x over previous
import functools

import jax
import jax.numpy as jnp
from jax import lax
from jax.experimental import pallas as pl
from jax.experimental.pallas import tpu as pltpu
from jax.experimental.pallas import tpu_sc as plsc

_B, _T, _D = 2, 2048, 1024
_H = 2048
_E = 8
_N = _B * _T

_BLK = 256
_NT = _N // _BLK
_NWK = _NT + _E - 1
_HBLK = 512
_NH = _H // _HBLK

_SC_NW = 32
_ROWS_PER_W = _N // _SC_NW
_CH = 64
_NCHUNK = _ROWS_PER_W // _CH


def _gate_body(x_ref, wg_ref, bg_ref, top1_ref):
    scores = jnp.dot(x_ref[...], wg_ref[...],
                     preferred_element_type=jnp.float32)
    scores = scores + bg_ref[...]
    best = jnp.max(scores, axis=-1, keepdims=True)
    lane = lax.broadcasted_iota(jnp.int32, scores.shape, 1)
    idx = jnp.where(scores == best, lane, _E)
    top1_ref[...] = jnp.min(idx, axis=-1, keepdims=True)


def _gate(x2d, Wg, bg2d):
    gblk = 512
    return pl.pallas_call(
        _gate_body,
        grid=(_N // gblk,),
        in_specs=[
            pl.BlockSpec((gblk, _D), lambda t: (t, 0)),
            pl.BlockSpec((_D, _E), lambda t: (0, 0)),
            pl.BlockSpec((1, _E), lambda t: (0, 0)),
        ],
        out_specs=pl.BlockSpec((gblk, 1), lambda t: (t, 0)),
        out_shape=jax.ShapeDtypeStruct((_N, 1), jnp.int32),
    )(x2d, Wg, bg2d)


def _metadata(top1):
    e_ids = jnp.arange(_E, dtype=jnp.int32)
    counts = jnp.sum((top1[:, None] == e_ids[None, :]).astype(jnp.int32),
                     axis=0)
    off = jnp.concatenate(
        [jnp.zeros((1,), jnp.int32), jnp.cumsum(counts)]).astype(jnp.int32)

    start_tile = off[:_E] // _BLK
    end_tile = (off[1:] + _BLK - 1) // _BLK
    tiles_e = jnp.where(counts > 0, end_tile - start_tile, 0)
    total = jnp.sum(tiles_e)

    work_e = jnp.repeat(e_ids, tiles_e, total_repeat_length=_NWK)
    i_arr = jnp.arange(_NWK, dtype=jnp.int32)
    valid = i_arr < total
    cum_excl = jnp.concatenate(
        [jnp.zeros((1,), jnp.int32), jnp.cumsum(tiles_e)[:-1]])
    last_e = work_e[jnp.maximum(total - 1, 0)]
    work_e = jnp.where(valid, work_e, last_e).astype(jnp.int32)
    work_t = start_tile[work_e] + (i_arr - cum_excl[work_e])
    work_t = jnp.clip(work_t, 0, _NT - 1).astype(jnp.int32)

    lo = jnp.maximum(off[work_e], work_t * _BLK)
    hi = jnp.minimum(off[work_e + 1], (work_t + 1) * _BLK)
    lo = jnp.where(valid, lo, 0).astype(jnp.int32)
    hi = jnp.where(valid, hi, 0).astype(jnp.int32)

    fv = jnp.concatenate(
        [jnp.ones((1,), jnp.bool_), work_t[1:] != work_t[:-1]])
    fv = fv.astype(jnp.int32)
    return work_t, work_e, lo, hi, fv


def _sc_gather(table, idx):
    mesh = plsc.VectorSubcoreMesh(core_axis_name="c", subcore_axis_name="s")

    @functools.partial(
        pl.kernel,
        mesh=mesh,
        out_type=jax.ShapeDtypeStruct((_N, _D), jnp.float32),
        scratch_types=[
            pltpu.VMEM((_CH,), jnp.int32),
            pltpu.VMEM((_CH, _D), jnp.float32),
            pltpu.SemaphoreType.DMA,
        ],
    )
    def k(table_hbm, idx_hbm, out_hbm, idx_v, rows_v, sem):
        wid = lax.axis_index("s") * 2 + lax.axis_index("c")
        base = wid * _ROWS_PER_W
        for c in range(_NCHUNK):
            b = base + c * _CH
            pltpu.sync_copy(idx_hbm.at[pl.ds(b, _CH)], idx_v)
            pltpu.async_copy(table_hbm.at[idx_v], rows_v, sem).wait()
            pltpu.sync_copy(rows_v, out_hbm.at[pl.ds(b, _CH), :])

    return k(table, idx)


def _ffn_body(wt, we, lo, hi, fv,
              xs_ref, w1_ref, b1_ref, w2_ref, b2_ref, out_ref):
    i = pl.program_id(0)
    j = pl.program_id(1)

    @pl.when(jnp.logical_and(fv[i] == 1, j == 0))
    def _zero():
        out_ref[...] = jnp.zeros_like(out_ref)

    @pl.when(lo[i] < hi[i])
    def _compute():
        x = xs_ref[...]
        h = jnp.dot(x, w1_ref[0], preferred_element_type=jnp.float32)
        h = jnp.maximum(h + b1_ref[0, 0][None, :], 0.0)
        y = jnp.dot(h, w2_ref[0], preferred_element_type=jnp.float32)
        y = y + jnp.where(j == _NH - 1, 1.0, 0.0) * b2_ref[0, 0][None, :]
        rows = wt[i] * _BLK + lax.broadcasted_iota(jnp.int32, (_BLK, 1), 0)
        mask = jnp.logical_and(rows >= lo[i], rows < hi[i])
        out_ref[...] += jnp.where(mask, y, 0.0)


def _ffn(meta, xs, W1, b1r, W2, b2r):
    grid_spec = pltpu.PrefetchScalarGridSpec(
        num_scalar_prefetch=5,
        grid=(_NWK, _NH),
        in_specs=[
            pl.BlockSpec((_BLK, _D),
                         lambda i, j, wt, we, lo, hi, fv: (wt[i], 0)),
            pl.BlockSpec((1, _D, _HBLK),
                         lambda i, j, wt, we, lo, hi, fv: (we[i], 0, j)),
            pl.BlockSpec((1, 1, _HBLK),
                         lambda i, j, wt, we, lo, hi, fv: (we[i], 0, j)),
            pl.BlockSpec((1, _HBLK, _D),
                         lambda i, j, wt, we, lo, hi, fv: (we[i], j, 0)),
            pl.BlockSpec((1, 1, _D),
                         lambda i, j, wt, we, lo, hi, fv: (we[i], 0, 0)),
        ],
        out_specs=pl.BlockSpec((_BLK, _D),
                               lambda i, j, wt, we, lo, hi, fv: (wt[i], 0)),
    )
    return pl.pallas_call(
        _ffn_body,
        grid_spec=grid_spec,
        out_shape=jax.ShapeDtypeStruct((_N, _D), jnp.float32),
        compiler_params=pltpu.CompilerParams(
            dimension_semantics=("arbitrary", "arbitrary")),
    )(*meta, xs, W1, b1r, W2, b2r)


@jax.jit
def kernel(x, Wg, bg, W1, b1, W2, b2):
    x2d = x.reshape(_N, _D)
    top1 = _gate(x2d, Wg, bg.reshape(1, _E))[:, 0]

    meta = _metadata(top1)
    sort_idx = jnp.argsort(top1).astype(jnp.int32)
    inv_idx = jnp.argsort(sort_idx).astype(jnp.int32)

    xs = _sc_gather(x2d, sort_idx)
    ys = _ffn(meta, xs, W1, b1.reshape(_E, 1, _H), W2, b2.reshape(_E, 1, _D))
    out = _sc_gather(ys, inv_idx)
    return out.reshape(_B, _T, _D)


# speedup vs baseline: 1.7119x; 1.0229x over previous
import functools

import jax
import jax.numpy as jnp
from jax import lax
from jax.experimental import pallas as pl
from jax.experimental.pallas import tpu as pltpu
from jax.experimental.pallas import tpu_sc as plsc

_B, _T, _D = 2, 2048, 1024
_H = 2048
_E = 8
_N = _B * _T

_BLK = 256
_NT = _N // _BLK
_NWK = _NT + _E - 1
_HBLK = 512
_NH = _H // _HBLK

_SC_NW = 32
_ROWS_PER_W = _N // _SC_NW
_CH = 64
_NCHUNK = _ROWS_PER_W // _CH


def _gate_body(x_ref, wg_ref, bg_ref, top1_ref):
    scores = jnp.dot(x_ref[...], wg_ref[...],
                     preferred_element_type=jnp.float32)
    scores = scores + bg_ref[...]
    best = jnp.max(scores, axis=-1, keepdims=True)
    lane = lax.broadcasted_iota(jnp.int32, scores.shape, 1)
    idx = jnp.where(scores == best, lane, _E)
    top1_ref[...] = jnp.min(idx, axis=-1, keepdims=True)


def _gate(x2d, Wg, bg2d):
    gblk = 512
    return pl.pallas_call(
        _gate_body,
        grid=(_N // gblk,),
        in_specs=[
            pl.BlockSpec((gblk, _D), lambda t: (t, 0)),
            pl.BlockSpec((_D, _E), lambda t: (0, 0)),
            pl.BlockSpec((1, _E), lambda t: (0, 0)),
        ],
        out_specs=pl.BlockSpec((gblk, 1), lambda t: (t, 0)),
        out_shape=jax.ShapeDtypeStruct((_N, 1), jnp.int32),
    )(x2d, Wg, bg2d)


def _metadata(top1):
    e_ids = jnp.arange(_E, dtype=jnp.int32)
    onehot = (top1[:, None] == e_ids[None, :]).astype(jnp.int32)
    within = jnp.cumsum(onehot, axis=0)
    counts = within[-1]
    off = jnp.concatenate(
        [jnp.zeros((1,), jnp.int32), jnp.cumsum(counts)]).astype(jnp.int32)
    pos = jnp.sum(onehot * (within - 1 + off[None, :_E]),
                  axis=1).astype(jnp.int32)

    start_tile = off[:_E] // _BLK
    end_tile = (off[1:] + _BLK - 1) // _BLK
    tiles_e = jnp.where(counts > 0, end_tile - start_tile, 0)
    total = jnp.sum(tiles_e)

    work_e = jnp.repeat(e_ids, tiles_e, total_repeat_length=_NWK)
    i_arr = jnp.arange(_NWK, dtype=jnp.int32)
    valid = i_arr < total
    cum_excl = jnp.concatenate(
        [jnp.zeros((1,), jnp.int32), jnp.cumsum(tiles_e)[:-1]])
    last_e = work_e[jnp.maximum(total - 1, 0)]
    work_e = jnp.where(valid, work_e, last_e).astype(jnp.int32)
    work_t = start_tile[work_e] + (i_arr - cum_excl[work_e])
    work_t = jnp.clip(work_t, 0, _NT - 1).astype(jnp.int32)

    lo = jnp.maximum(off[work_e], work_t * _BLK)
    hi = jnp.minimum(off[work_e + 1], (work_t + 1) * _BLK)
    lo = jnp.where(valid, lo, 0).astype(jnp.int32)
    hi = jnp.where(valid, hi, 0).astype(jnp.int32)

    fv = jnp.concatenate(
        [jnp.ones((1,), jnp.bool_), work_t[1:] != work_t[:-1]])
    fv = fv.astype(jnp.int32)
    return (work_t, work_e, lo, hi, fv), pos


def _sc_gather(table, idx):
    mesh = plsc.VectorSubcoreMesh(core_axis_name="c", subcore_axis_name="s")

    @functools.partial(
        pl.kernel,
        mesh=mesh,
        out_type=jax.ShapeDtypeStruct((_N, _D), jnp.float32),
        scratch_types=[
            pltpu.VMEM((_CH,), jnp.int32),
            pltpu.VMEM((_CH, _D), jnp.float32),
            pltpu.SemaphoreType.DMA,
        ],
    )
    def k(table_hbm, idx_hbm, out_hbm, idx_v, rows_v, sem):
        wid = lax.axis_index("s") * 2 + lax.axis_index("c")
        base = wid * _ROWS_PER_W
        for c in range(_NCHUNK):
            b = base + c * _CH
            pltpu.sync_copy(idx_hbm.at[pl.ds(b, _CH)], idx_v)
            pltpu.async_copy(table_hbm.at[idx_v], rows_v, sem).wait()
            pltpu.sync_copy(rows_v, out_hbm.at[pl.ds(b, _CH), :])

    return k(table, idx)


def _sc_scatter(rows, idx):
    mesh = plsc.VectorSubcoreMesh(core_axis_name="c", subcore_axis_name="s")

    @functools.partial(
        pl.kernel,
        mesh=mesh,
        out_type=jax.ShapeDtypeStruct((_N, _D), jnp.float32),
        scratch_types=[
            pltpu.VMEM((_CH,), jnp.int32),
            pltpu.VMEM((_CH, _D), jnp.float32),
            pltpu.SemaphoreType.DMA,
        ],
    )
    def k(rows_hbm, idx_hbm, out_hbm, idx_v, rows_v, sem):
        wid = lax.axis_index("s") * 2 + lax.axis_index("c")
        base = wid * _ROWS_PER_W
        for c in range(_NCHUNK):
            b = base + c * _CH
            pltpu.sync_copy(idx_hbm.at[pl.ds(b, _CH)], idx_v)
            pltpu.sync_copy(rows_hbm.at[pl.ds(b, _CH)], rows_v)
            pltpu.async_copy(rows_v, out_hbm.at[idx_v], sem).wait()

    return k(rows, idx)


def _ffn_body(wt, we, lo, hi, fv,
              xs_ref, w1_ref, b1_ref, w2_ref, b2_ref, out_ref):
    i = pl.program_id(0)
    j = pl.program_id(1)

    @pl.when(jnp.logical_and(fv[i] == 1, j == 0))
    def _zero():
        out_ref[...] = jnp.zeros_like(out_ref)

    @pl.when(lo[i] < hi[i])
    def _compute():
        x = xs_ref[...]
        h = jnp.dot(x, w1_ref[0], preferred_element_type=jnp.float32)
        h = jnp.maximum(h + b1_ref[0, 0][None, :], 0.0)
        y = jnp.dot(h, w2_ref[0], preferred_element_type=jnp.float32)
        y = y + jnp.where(j == _NH - 1, 1.0, 0.0) * b2_ref[0, 0][None, :]
        rows = wt[i] * _BLK + lax.broadcasted_iota(jnp.int32, (_BLK, 1), 0)
        mask = jnp.logical_and(rows >= lo[i], rows < hi[i])
        out_ref[...] += jnp.where(mask, y, 0.0)


def _ffn(meta, xs, W1, b1r, W2, b2r):
    grid_spec = pltpu.PrefetchScalarGridSpec(
        num_scalar_prefetch=5,
        grid=(_NWK, _NH),
        in_specs=[
            pl.BlockSpec((_BLK, _D),
                         lambda i, j, wt, we, lo, hi, fv: (wt[i], 0)),
            pl.BlockSpec((1, _D, _HBLK),
                         lambda i, j, wt, we, lo, hi, fv: (we[i], 0, j)),
            pl.BlockSpec((1, 1, _HBLK),
                         lambda i, j, wt, we, lo, hi, fv: (we[i], 0, j)),
            pl.BlockSpec((1, _HBLK, _D),
                         lambda i, j, wt, we, lo, hi, fv: (we[i], j, 0)),
            pl.BlockSpec((1, 1, _D),
                         lambda i, j, wt, we, lo, hi, fv: (we[i], 0, 0)),
        ],
        out_specs=pl.BlockSpec((_BLK, _D),
                               lambda i, j, wt, we, lo, hi, fv: (wt[i], 0)),
    )
    return pl.pallas_call(
        _ffn_body,
        grid_spec=grid_spec,
        out_shape=jax.ShapeDtypeStruct((_N, _D), jnp.float32),
        compiler_params=pltpu.CompilerParams(
            dimension_semantics=("arbitrary", "arbitrary")),
    )(*meta, xs, W1, b1r, W2, b2r)


@jax.jit
def kernel(x, Wg, bg, W1, b1, W2, b2):
    x2d = x.reshape(_N, _D)
    top1 = _gate(x2d, Wg, bg.reshape(1, _E))[:, 0]

    meta, pos = _metadata(top1)

    xs = _sc_scatter(x2d, pos)
    ys = _ffn(meta, xs, W1, b1.reshape(_E, 1, _H), W2, b2.reshape(_E, 1, _D))
    out = _sc_gather(ys, pos)
    return out.reshape(_B, _T, _D)


# speedup vs baseline: 2.4586x; 1.4362x over previous
import functools

import jax
import jax.numpy as jnp
from jax import lax
from jax.experimental import pallas as pl
from jax.experimental.pallas import tpu as pltpu
from jax.experimental.pallas import tpu_sc as plsc

_B, _T, _D = 2, 2048, 1024
_H = 2048
_E = 8
_N = _B * _T

_BLK = 256
_NT = _N // _BLK
_NWK = _NT + _E - 1

_SC_NW = 32
_ROWS_PER_W = _N // _SC_NW
_CH = 64
_NCHUNK = _ROWS_PER_W // _CH


def _gate_body(x_ref, wg_ref, bg_ref, top1_ref):
    scores = jnp.dot(x_ref[...], wg_ref[...],
                     preferred_element_type=jnp.float32)
    scores = scores + bg_ref[...]
    best = jnp.max(scores, axis=-1, keepdims=True)
    lane = lax.broadcasted_iota(jnp.int32, scores.shape, 1)
    idx = jnp.where(scores == best, lane, _E)
    top1_ref[...] = jnp.min(idx, axis=-1, keepdims=True)


def _gate(x2d, Wg, bg2d):
    gblk = 512
    return pl.pallas_call(
        _gate_body,
        grid=(_N // gblk,),
        in_specs=[
            pl.BlockSpec((gblk, _D), lambda t: (t, 0)),
            pl.BlockSpec((_D, _E), lambda t: (0, 0)),
            pl.BlockSpec((1, _E), lambda t: (0, 0)),
        ],
        out_specs=pl.BlockSpec((gblk, 1), lambda t: (t, 0)),
        out_shape=jax.ShapeDtypeStruct((_N, 1), jnp.int32),
    )(x2d, Wg, bg2d)


def _metadata(top1):
    e_ids = jnp.arange(_E, dtype=jnp.int32)
    onehot = (top1[:, None] == e_ids[None, :]).astype(jnp.int32)
    within = jnp.cumsum(onehot, axis=0)
    counts = within[-1]
    off = jnp.concatenate(
        [jnp.zeros((1,), jnp.int32), jnp.cumsum(counts)]).astype(jnp.int32)
    pos = jnp.sum(onehot * (within - 1 + off[None, :_E]),
                  axis=1).astype(jnp.int32)

    start_tile = off[:_E] // _BLK
    end_tile = (off[1:] + _BLK - 1) // _BLK
    tiles_e = jnp.where(counts > 0, end_tile - start_tile, 0)
    total = jnp.sum(tiles_e)

    work_e = jnp.repeat(e_ids, tiles_e, total_repeat_length=_NWK)
    i_arr = jnp.arange(_NWK, dtype=jnp.int32)
    valid = i_arr < total
    cum_excl = jnp.concatenate(
        [jnp.zeros((1,), jnp.int32), jnp.cumsum(tiles_e)[:-1]])
    last_e = work_e[jnp.maximum(total - 1, 0)]
    work_e = jnp.where(valid, work_e, last_e).astype(jnp.int32)
    work_t = start_tile[work_e] + (i_arr - cum_excl[work_e])
    work_t = jnp.clip(work_t, 0, _NT - 1).astype(jnp.int32)

    lo = jnp.maximum(off[work_e], work_t * _BLK)
    hi = jnp.minimum(off[work_e + 1], (work_t + 1) * _BLK)
    lo = jnp.where(valid, lo, 0).astype(jnp.int32)
    hi = jnp.where(valid, hi, 0).astype(jnp.int32)

    fv = jnp.concatenate(
        [jnp.ones((1,), jnp.bool_), work_t[1:] != work_t[:-1]])
    fv = fv.astype(jnp.int32)
    return (work_t, work_e, lo, hi, fv), pos


def _sc_gather(table, idx):
    mesh = plsc.VectorSubcoreMesh(core_axis_name="c", subcore_axis_name="s")

    @functools.partial(
        pl.kernel,
        mesh=mesh,
        out_type=jax.ShapeDtypeStruct((_N, _D), jnp.float32),
        scratch_types=[
            pltpu.VMEM((_CH,), jnp.int32),
            pltpu.VMEM((_CH, _D), jnp.float32),
            pltpu.SemaphoreType.DMA,
        ],
    )
    def k(table_hbm, idx_hbm, out_hbm, idx_v, rows_v, sem):
        wid = lax.axis_index("s") * 2 + lax.axis_index("c")
        base = wid * _ROWS_PER_W
        for c in range(_NCHUNK):
            b = base + c * _CH
            pltpu.sync_copy(idx_hbm.at[pl.ds(b, _CH)], idx_v)
            pltpu.async_copy(table_hbm.at[idx_v], rows_v, sem).wait()
            pltpu.sync_copy(rows_v, out_hbm.at[pl.ds(b, _CH), :])

    return k(table, idx)


def _sc_scatter(rows, idx):
    mesh = plsc.VectorSubcoreMesh(core_axis_name="c", subcore_axis_name="s")

    @functools.partial(
        pl.kernel,
        mesh=mesh,
        out_type=jax.ShapeDtypeStruct((_N, _D), jnp.float32),
        scratch_types=[
            pltpu.VMEM((_CH,), jnp.int32),
            pltpu.VMEM((_CH, _D), jnp.float32),
            pltpu.SemaphoreType.DMA,
        ],
    )
    def k(rows_hbm, idx_hbm, out_hbm, idx_v, rows_v, sem):
        wid = lax.axis_index("s") * 2 + lax.axis_index("c")
        base = wid * _ROWS_PER_W
        for c in range(_NCHUNK):
            b = base + c * _CH
            pltpu.sync_copy(idx_hbm.at[pl.ds(b, _CH)], idx_v)
            pltpu.sync_copy(rows_hbm.at[pl.ds(b, _CH)], rows_v)
            pltpu.async_copy(rows_v, out_hbm.at[idx_v], sem).wait()

    return k(rows, idx)


def _ffn_body(wt, we, lo, hi, fv,
              xs_ref, w1_ref, b1_ref, w2_ref, b2_ref, out_ref):
    i = pl.program_id(0)

    @pl.when(fv[i] == 1)
    def _zero():
        out_ref[...] = jnp.zeros_like(out_ref)

    @pl.when(lo[i] < hi[i])
    def _compute():
        x = xs_ref[...]
        h = jnp.dot(x, w1_ref[0], preferred_element_type=jnp.float32)
        h = jnp.maximum(h + b1_ref[0, 0][None, :], 0.0)
        y = jnp.dot(h, w2_ref[0], preferred_element_type=jnp.float32)
        y = y + b2_ref[0, 0][None, :]
        rows = wt[i] * _BLK + lax.broadcasted_iota(jnp.int32, (_BLK, 1), 0)
        mask = jnp.logical_and(rows >= lo[i], rows < hi[i])
        out_ref[...] += jnp.where(mask, y, 0.0)


def _ffn(meta, xs, W1, b1r, W2, b2r):
    grid_spec = pltpu.PrefetchScalarGridSpec(
        num_scalar_prefetch=5,
        grid=(_NWK,),
        in_specs=[
            pl.BlockSpec((_BLK, _D),
                         lambda i, wt, we, lo, hi, fv: (wt[i], 0)),
            pl.BlockSpec((1, _D, _H),
                         lambda i, wt, we, lo, hi, fv: (we[i], 0, 0)),
            pl.BlockSpec((1, 1, _H),
                         lambda i, wt, we, lo, hi, fv: (we[i], 0, 0)),
            pl.BlockSpec((1, _H, _D),
                         lambda i, wt, we, lo, hi, fv: (we[i], 0, 0)),
            pl.BlockSpec((1, 1, _D),
                         lambda i, wt, we, lo, hi, fv: (we[i], 0, 0)),
        ],
        out_specs=pl.BlockSpec((_BLK, _D),
                               lambda i, wt, we, lo, hi, fv: (wt[i], 0)),
    )
    return pl.pallas_call(
        _ffn_body,
        grid_spec=grid_spec,
        out_shape=jax.ShapeDtypeStruct((_N, _D), jnp.float32),
        compiler_params=pltpu.CompilerParams(
            dimension_semantics=("arbitrary",)),
    )(*meta, xs, W1, b1r, W2, b2r)


@jax.jit
def kernel(x, Wg, bg, W1, b1, W2, b2):
    x2d = x.reshape(_N, _D)
    top1 = _gate(x2d, Wg, bg.reshape(1, _E))[:, 0]

    meta, pos = _metadata(top1)

    xs = _sc_scatter(x2d, pos)
    ys = _ffn(meta, xs, W1, b1.reshape(_E, 1, _H), W2, b2.reshape(_E, 1, _D))
    out = _sc_gather(ys, pos)
    return out.reshape(_B, _T, _D)
